# baseline (speedup 1.0000x reference)
import jax
import jax.numpy as jnp
from jax import lax
from jax.experimental import pallas as pl
from jax.experimental.pallas import tpu as pltpu
from jax.experimental.pallas import tpu_sc as plsc

_NW = 32
_BM = 1024


def _logits_body(x_ref, wt_ref, b_ref, out_ref):
    out_ref[...] = lax.dot_general(
        x_ref[...], wt_ref[...], (((1,), (0,)), ((), ())),
        preferred_element_type=jnp.float32) + b_ref[...]


def _router_body(logits_hbm, idx_hbm, wts_hbm, lbuf, iout, wout):
    tokens = logits_hbm.shape[0]
    experts = logits_hbm.shape[1]
    chunk = tokens // _NW
    wid = lax.axis_index("s") * 2 + lax.axis_index("c")
    base = wid * chunk
    pltpu.sync_copy(logits_hbm.at[pl.ds(base, chunk)], lbuf)
    lane = lax.iota(jnp.int32, 16)

    def group(g, carry):
        rows = lane + g * 16
        zero = jnp.zeros((16,), jnp.int32)
        one = jnp.ones((16,), jnp.int32)
        v0 = plsc.load_gather(lbuf, [rows, zero])
        v1 = plsc.load_gather(lbuf, [rows, one])
        sw = v1 > v0
        m1 = jnp.where(sw, v1, v0)
        i1 = jnp.where(sw, one, zero)
        m2 = jnp.where(sw, v0, v1)
        i2 = jnp.where(sw, zero, one)
        for e in range(2, experts):
            ev = jnp.full((16,), e, jnp.int32)
            v = plsc.load_gather(lbuf, [rows, ev])
            gt1 = v > m1
            gt2 = v > m2
            m2 = jnp.where(gt1, m1, jnp.where(gt2, v, m2))
            i2 = jnp.where(gt1, i1, jnp.where(gt2, ev, i2))
            m1 = jnp.where(gt1, v, m1)
            i1 = jnp.where(gt1, ev, i1)
        ex = jnp.exp(m2 - m1)
        w1 = 1.0 / (1.0 + ex)
        pos = rows * 2
        plsc.store_scatter(iout, [pos], i1)
        plsc.store_scatter(iout, [pos + 1], i2)
        plsc.store_scatter(wout, [pos], w1)
        plsc.store_scatter(wout, [pos + 1], 1.0 - w1)
        return carry

    lax.fori_loop(0, chunk // 16, group, 0)
    pltpu.sync_copy(iout, idx_hbm.at[pl.ds(base * 2, chunk * 2)])
    pltpu.sync_copy(wout, wts_hbm.at[pl.ds(base * 2, chunk * 2)])


def kernel(x, W, b):
    batch, seq, feat = x.shape
    tokens = batch * seq
    experts = W.shape[0]
    xf = x.reshape(tokens, feat)

    logits = pl.pallas_call(
        _logits_body,
        grid=(tokens // _BM,),
        in_specs=[
            pl.BlockSpec((_BM, feat), lambda i: (i, 0)),
            pl.BlockSpec((feat, experts), lambda i: (0, 0)),
            pl.BlockSpec((1, experts), lambda i: (0, 0)),
        ],
        out_specs=pl.BlockSpec((_BM, experts), lambda i: (i, 0)),
        out_shape=jax.ShapeDtypeStruct((tokens, experts), jnp.float32),
    )(xf, W.T, b.reshape(1, experts))

    chunk = tokens // _NW
    router = pl.kernel(
        _router_body,
        out_type=(jax.ShapeDtypeStruct((tokens * 2,), jnp.int32),
                  jax.ShapeDtypeStruct((tokens * 2,), jnp.float32)),
        mesh=plsc.VectorSubcoreMesh(core_axis_name="c", subcore_axis_name="s"),
        compiler_params=pltpu.CompilerParams(
            needs_layout_passes=False, use_tc_tiling_on_sc=False),
        scratch_types=[
            pltpu.VMEM((chunk, experts), jnp.float32),
            pltpu.VMEM((chunk * 2,), jnp.int32),
            pltpu.VMEM((chunk * 2,), jnp.float32),
        ],
    )
    idx, wts = router(logits)
    return idx.reshape(batch, seq, 2), wts.reshape(batch, seq, 2)


# speedup vs baseline: 1.5560x; 1.5560x over previous
import jax
import jax.numpy as jnp
from jax import lax
from jax.experimental import pallas as pl
from jax.experimental.pallas import tpu as pltpu
from jax.experimental.pallas import tpu_sc as plsc

_NW = 32
_BM = 1024


def _logits_body(x_ref, wt_ref, b_ref, out_ref):
    lgt = lax.dot_general(
        x_ref[...], wt_ref[...], (((1,), (0,)), ((), ())),
        preferred_element_type=jnp.float32) + b_ref[...]
    experts = lgt.shape[1]
    for j in range(lgt.shape[0] // 128):
        out_ref[pl.ds(j * experts, experts), :] = lgt[j * 128:(j + 1) * 128, :].T


def _router_body(logits_hbm, i1_hbm, i2_hbm, w1_hbm, w2_hbm,
                 lbuf, i1b, i2b, w1b, w2b):
    flat = logits_hbm.shape[0]
    experts = 16
    chunk = flat // experts // _NW
    wid = lax.axis_index("s") * 2 + lax.axis_index("c")
    base = wid * chunk
    pltpu.sync_copy(logits_hbm.at[pl.ds(base * experts, chunk * experts)], lbuf)
    lane = lax.iota(jnp.int32, 16)

    def group(g, carry):
        gbase = lane + (lax.div(g, 8) * 2048 + lax.rem(g, 8) * 16)
        zero = jnp.zeros((16,), jnp.int32)
        one = jnp.ones((16,), jnp.int32)
        v0 = plsc.load_gather(lbuf, [gbase])
        v1 = plsc.load_gather(lbuf, [gbase + 128])
        sw = v1 > v0
        m1 = jnp.where(sw, v1, v0)
        i1 = jnp.where(sw, one, zero)
        m2 = jnp.where(sw, v0, v1)
        i2 = jnp.where(sw, zero, one)
        for e in range(2, experts):
            ev = jnp.full((16,), e, jnp.int32)
            v = plsc.load_gather(lbuf, [gbase + e * 128])
            gt1 = v > m1
            gt2 = v > m2
            m2 = jnp.where(gt1, m1, jnp.where(gt2, v, m2))
            i2 = jnp.where(gt1, i1, jnp.where(gt2, ev, i2))
            m1 = jnp.where(gt1, v, m1)
            i1 = jnp.where(gt1, ev, i1)
        ex = jnp.exp(m2 - m1)
        w1 = 1.0 / (1.0 + ex)
        sl = pl.ds(g * 16, 16)
        i1b[sl] = i1
        i2b[sl] = i2
        w1b[sl] = w1
        w2b[sl] = 1.0 - w1
        return carry

    lax.fori_loop(0, chunk // 16, group, 0)
    pltpu.sync_copy(i1b, i1_hbm.at[pl.ds(base, chunk)])
    pltpu.sync_copy(i2b, i2_hbm.at[pl.ds(base, chunk)])
    pltpu.sync_copy(w1b, w1_hbm.at[pl.ds(base, chunk)])
    pltpu.sync_copy(w2b, w2_hbm.at[pl.ds(base, chunk)])


def kernel(x, W, b):
    batch, seq, feat = x.shape
    tokens = batch * seq
    experts = W.shape[0]
    xf = x.reshape(tokens, feat)
    pack = 128 // experts

    logits8 = pl.pallas_call(
        _logits_body,
        grid=(tokens // _BM,),
        in_specs=[
            pl.BlockSpec((_BM, feat), lambda i: (i, 0)),
            pl.BlockSpec((feat, experts), lambda i: (0, 0)),
            pl.BlockSpec((1, experts), lambda i: (0, 0)),
        ],
        out_specs=pl.BlockSpec((_BM // pack, 128), lambda i: (i, 0)),
        out_shape=jax.ShapeDtypeStruct((tokens // pack, 128), jnp.float32),
    )(xf, W.T, b.reshape(1, experts))

    chunk = tokens // _NW
    router = pl.kernel(
        _router_body,
        out_type=(jax.ShapeDtypeStruct((tokens,), jnp.int32),
                  jax.ShapeDtypeStruct((tokens,), jnp.int32),
                  jax.ShapeDtypeStruct((tokens,), jnp.float32),
                  jax.ShapeDtypeStruct((tokens,), jnp.float32)),
        mesh=plsc.VectorSubcoreMesh(core_axis_name="c", subcore_axis_name="s"),
        compiler_params=pltpu.CompilerParams(needs_layout_passes=False),
        scratch_types=[
            pltpu.VMEM((chunk * experts,), jnp.float32),
            pltpu.VMEM((chunk,), jnp.int32),
            pltpu.VMEM((chunk,), jnp.int32),
            pltpu.VMEM((chunk,), jnp.float32),
            pltpu.VMEM((chunk,), jnp.float32),
        ],
    )
    i1, i2, w1, w2 = router(logits8.reshape(tokens * experts))
    idx = jnp.stack([i1, i2], axis=-1).reshape(batch, seq, 2)
    wts = jnp.stack([w1, w2], axis=-1).reshape(batch, seq, 2)
    return idx, wts


# speedup vs baseline: 1.6285x; 1.0466x over previous
import jax
import jax.numpy as jnp
from jax import lax
from jax.experimental import pallas as pl
from jax.experimental.pallas import tpu as pltpu
from jax.experimental.pallas import tpu_sc as plsc

_NW = 32
_BM = 1024


def _logits_body(x_ref, wt_ref, b_ref, out_ref):
    lgt = lax.dot_general(
        x_ref[...], wt_ref[...], (((1,), (1,)), ((), ())),
        preferred_element_type=jnp.float32) + b_ref[...]
    experts = lgt.shape[1]
    for j in range(lgt.shape[0] // 128):
        out_ref[pl.ds(j * experts, experts), :] = lgt[j * 128:(j + 1) * 128, :].T


def _router_body(logits_hbm, i1_hbm, i2_hbm, w1_hbm, w2_hbm,
                 lbuf, i1b, i2b, w1b, w2b):
    flat = logits_hbm.shape[0]
    experts = 16
    chunk = flat // experts // _NW
    wid = lax.axis_index("s") * 2 + lax.axis_index("c")
    base = wid * chunk
    pltpu.sync_copy(logits_hbm.at[pl.ds(base * experts, chunk * experts)], lbuf)
    lane = lax.iota(jnp.int32, 16)

    def group(g, carry):
        gbase = lane + (lax.div(g, 8) * 2048 + lax.rem(g, 8) * 16)
        zero = jnp.zeros((16,), jnp.int32)
        one = jnp.ones((16,), jnp.int32)
        v0 = plsc.load_gather(lbuf, [gbase])
        v1 = plsc.load_gather(lbuf, [gbase + 128])
        sw = v1 > v0
        m1 = jnp.where(sw, v1, v0)
        i1 = jnp.where(sw, one, zero)
        m2 = jnp.where(sw, v0, v1)
        i2 = jnp.where(sw, zero, one)
        for e in range(2, experts):
            ev = jnp.full((16,), e, jnp.int32)
            v = plsc.load_gather(lbuf, [gbase + e * 128])
            gt1 = v > m1
            gt2 = v > m2
            m2 = jnp.where(gt1, m1, jnp.where(gt2, v, m2))
            i2 = jnp.where(gt1, i1, jnp.where(gt2, ev, i2))
            m1 = jnp.where(gt1, v, m1)
            i1 = jnp.where(gt1, ev, i1)
        ex = jnp.exp(m2 - m1)
        w1 = 1.0 / (1.0 + ex)
        sl = pl.ds(g * 16, 16)
        i1b[sl] = i1
        i2b[sl] = i2
        w1b[sl] = w1
        w2b[sl] = 1.0 - w1
        return carry

    lax.fori_loop(0, chunk // 16, group, 0)
    pltpu.sync_copy(i1b, i1_hbm.at[pl.ds(base, chunk)])
    pltpu.sync_copy(i2b, i2_hbm.at[pl.ds(base, chunk)])
    pltpu.sync_copy(w1b, w1_hbm.at[pl.ds(base, chunk)])
    pltpu.sync_copy(w2b, w2_hbm.at[pl.ds(base, chunk)])


def kernel(x, W, b):
    batch, seq, feat = x.shape
    tokens = batch * seq
    experts = W.shape[0]
    xf = x.reshape(tokens, feat)
    pack = 128 // experts

    logits8 = pl.pallas_call(
        _logits_body,
        grid=(tokens // _BM,),
        in_specs=[
            pl.BlockSpec((_BM, feat), lambda i: (i, 0)),
            pl.BlockSpec((experts, feat), lambda i: (0, 0)),
            pl.BlockSpec((1, experts), lambda i: (0, 0)),
        ],
        out_specs=pl.BlockSpec((_BM // pack, 128), lambda i: (i, 0)),
        out_shape=jax.ShapeDtypeStruct((tokens // pack, 128), jnp.float32),
    )(xf, W, b.reshape(1, experts))

    chunk = tokens // _NW
    router = pl.kernel(
        _router_body,
        out_type=(jax.ShapeDtypeStruct((tokens,), jnp.int32),
                  jax.ShapeDtypeStruct((tokens,), jnp.int32),
                  jax.ShapeDtypeStruct((tokens,), jnp.float32),
                  jax.ShapeDtypeStruct((tokens,), jnp.float32)),
        mesh=plsc.VectorSubcoreMesh(core_axis_name="c", subcore_axis_name="s"),
        compiler_params=pltpu.CompilerParams(needs_layout_passes=False),
        scratch_types=[
            pltpu.VMEM((chunk * experts,), jnp.float32),
            pltpu.VMEM((chunk,), jnp.int32),
            pltpu.VMEM((chunk,), jnp.int32),
            pltpu.VMEM((chunk,), jnp.float32),
            pltpu.VMEM((chunk,), jnp.float32),
        ],
    )
    i1, i2, w1, w2 = router(logits8.reshape(tokens * experts))
    idx = jnp.stack([i1, i2], axis=-1).reshape(batch, seq, 2)
    wts = jnp.stack([w1, w2], axis=-1).reshape(batch, seq, 2)
    return idx, wts
